# Initial kernel scaffold; baseline (speedup 1.0000x reference)
import jax
import jax.numpy as jnp
from jax.experimental import pallas as pl


def kernel(b, m_indices, m_values, W1, b1, W2, b2, W3, b3):
    raise NotImplementedError("write your pallas kernel here")


# speedup vs baseline: 92.4195x; 92.4195x over previous
import functools

import jax
import jax.numpy as jnp
from jax import lax
from jax.experimental import pallas as pl
from jax.experimental.pallas import tpu as pltpu
from jax.experimental.pallas import tpu_sc as plsc

NC = 2
NS = 16
NW = NC * NS
L = 16
C = 2048


def _edge_partition(nnz):
    e_per = pl.cdiv(nnz, NW * C) * C
    return e_per, NW * e_per


def _make_diag_kernel(n, e_per):
    mesh = plsc.VectorSubcoreMesh(core_axis_name="c", subcore_axis_name="s",
                                  num_cores=NC, num_subcores=NS)
    nch = e_per // C

    @functools.partial(
        pl.kernel,
        out_type=(
            jax.ShapeDtypeStruct((NW, n), jnp.float32),
            jax.ShapeDtypeStruct((NW, n), jnp.int32),
            jax.ShapeDtypeStruct((NW, L), jnp.float32),
        ),
        mesh=mesh,
        scratch_types=[
            pltpu.VMEM((n,), jnp.float32),
            pltpu.VMEM((n,), jnp.int32),
            pltpu.VMEM((C,), jnp.int32),
            pltpu.VMEM((C,), jnp.int32),
            pltpu.VMEM((C,), jnp.float32),
            pltpu.VMEM((L,), jnp.float32),
        ],
        compiler_params=pltpu.CompilerParams(needs_layout_passes=False),
    )
    def diag_kernel(rows_h, cols_h, vals_h, diag_o, pos_o, max_o,
                    diag_v, pos_v, rbuf, cbuf, vbuf, max_v):
        wid = lax.axis_index("s") * NC + lax.axis_index("c")
        base = wid * e_per

        def init_body(i, carry):
            diag_v[pl.ds(i * L, L)] = jnp.zeros((L,), jnp.float32)
            pos_v[pl.ds(i * L, L)] = jnp.full((L,), -1, jnp.int32)
            return carry

        lax.fori_loop(0, n // L, init_body, 0)

        iota = lax.iota(jnp.int32, L)

        def chunk_body(ch, maxcarry):
            start = base + ch * C
            pltpu.sync_copy(rows_h.at[pl.ds(start, C)], rbuf)
            pltpu.sync_copy(cols_h.at[pl.ds(start, C)], cbuf)
            pltpu.sync_copy(vals_h.at[pl.ds(start, C)], vbuf)

            def vec_body(v, mc):
                r16 = rbuf[pl.ds(v * L, L)]
                c16 = cbuf[pl.ds(v * L, L)]
                v16 = vbuf[pl.ds(v * L, L)]
                m = r16 == c16
                plsc.store_scatter(diag_v, [r16], v16, mask=m)
                pos16 = iota + (start + v * L)
                plsc.store_scatter(pos_v, [r16], pos16, mask=m)
                return jnp.maximum(mc, jnp.abs(v16))

            return lax.fori_loop(0, C // L, vec_body, maxcarry)

        maxv = lax.fori_loop(0, nch, chunk_body, jnp.zeros((L,), jnp.float32))
        max_v[...] = maxv
        pltpu.sync_copy(diag_v, diag_o.at[wid])
        pltpu.sync_copy(pos_v, pos_o.at[wid])
        pltpu.sync_copy(max_v, max_o.at[wid])

    return diag_kernel


def _mlp_body(diag_ref, pos_ref, maxabs_ref, b_ref, W1b_ref, b1_ref,
              W2_ref, b2_ref, W3_ref, b3_ref, y_ref):
    m_max = jnp.maximum(jnp.max(maxabs_ref[...]), jnp.float32(1e-16))
    pos = pos_ref[...]
    val = diag_ref[...]
    maxpos = jnp.max(pos, axis=0)
    diag = jnp.sum(jnp.where(pos == maxpos[None, :], val, 0.0), axis=0)
    f1 = b_ref[0, :] / m_max
    xf = jnp.stack([f1, diag], axis=1)
    h1 = jnp.maximum(
        jnp.dot(xf, W1b_ref[...], preferred_element_type=jnp.float32)
        + b1_ref[0, :], 0.0)
    h2 = jnp.maximum(
        jnp.dot(h1, W2_ref[...], preferred_element_type=jnp.float32)
        + b2_ref[0, :], 0.0)
    y = jnp.dot(h2, W3_ref[...], preferred_element_type=jnp.float32)[:, 0]
    y_ref[0, :] = y + b3_ref[0, 0]


def _make_mlp_call(n, h, r_blk):
    g = n // r_blk
    return pl.pallas_call(
        _mlp_body,
        grid=(g,),
        in_specs=[
            pl.BlockSpec((NW, r_blk), lambda i: (0, i)),
            pl.BlockSpec((NW, r_blk), lambda i: (0, i)),
            pl.BlockSpec((NW, L), lambda i: (0, 0)),
            pl.BlockSpec((1, r_blk), lambda i: (0, i)),
            pl.BlockSpec((2, h), lambda i: (0, 0)),
            pl.BlockSpec((1, h), lambda i: (0, 0)),
            pl.BlockSpec((h, h), lambda i: (0, 0)),
            pl.BlockSpec((1, h), lambda i: (0, 0)),
            pl.BlockSpec((h, 1), lambda i: (0, 0)),
            pl.BlockSpec((1, 1), lambda i: (0, 0)),
        ],
        out_specs=pl.BlockSpec((1, r_blk), lambda i: (0, i)),
        out_shape=jax.ShapeDtypeStruct((1, n), jnp.float32),
        compiler_params=pltpu.CompilerParams(
            dimension_semantics=("arbitrary",)),
    )


def _make_spmv_kernel(n, e_per):
    mesh = plsc.VectorSubcoreMesh(core_axis_name="c", subcore_axis_name="s",
                                  num_cores=NC, num_subcores=NS)
    nch = e_per // C

    @functools.partial(
        pl.kernel,
        out_type=jax.ShapeDtypeStruct((NW, n), jnp.float32),
        mesh=mesh,
        scratch_types=[
            pltpu.VMEM((n,), jnp.float32),
            pltpu.VMEM((n,), jnp.float32),
            pltpu.VMEM((C,), jnp.int32),
            pltpu.VMEM((C,), jnp.int32),
            pltpu.VMEM((C,), jnp.float32),
        ],
        compiler_params=pltpu.CompilerParams(needs_layout_passes=False),
    )
    def spmv_kernel(rows_h, cols_h, vals_h, y_h, p_o,
                    y_v, p_v, rbuf, cbuf, vbuf):
        wid = lax.axis_index("s") * NC + lax.axis_index("c")
        base = wid * e_per
        pltpu.sync_copy(y_h, y_v)

        def init_body(i, carry):
            p_v[pl.ds(i * L, L)] = jnp.zeros((L,), jnp.float32)
            return carry

        lax.fori_loop(0, n // L, init_body, 0)

        def chunk_body(ch, carry):
            start = base + ch * C
            pltpu.sync_copy(rows_h.at[pl.ds(start, C)], rbuf)
            pltpu.sync_copy(cols_h.at[pl.ds(start, C)], cbuf)
            pltpu.sync_copy(vals_h.at[pl.ds(start, C)], vbuf)

            def vec_body(v, c):
                r16 = rbuf[pl.ds(v * L, L)]
                c16 = cbuf[pl.ds(v * L, L)]
                v16 = vbuf[pl.ds(v * L, L)]
                yv = plsc.load_gather(y_v, [c16])
                plsc.addupdate_scatter(p_v, [r16], v16 * yv)
                return c

            return lax.fori_loop(0, C // L, vec_body, carry)

        lax.fori_loop(0, nch, chunk_body, 0)
        pltpu.sync_copy(p_v, p_o.at[wid])

    return spmv_kernel


def _scale_body(p_ref, b_ref, y_ref, out_ref):
    p = jnp.sum(p_ref[...], axis=0)
    pp = jnp.sum(p * p)
    bp = jnp.sum(p * b_ref[0, :])
    scaler = jnp.maximum(bp / jnp.maximum(pp, jnp.float32(1e-16)),
                         jnp.float32(1e-16))
    out_ref[0, :] = y_ref[0, :] * scaler


def _make_scale_call(n):
    return pl.pallas_call(
        _scale_body,
        out_shape=jax.ShapeDtypeStruct((1, n), jnp.float32),
    )


def kernel(b, m_indices, m_values, W1, b1, W2, b2, W3, b3):
    n = b.shape[0]
    nnz = m_values.shape[0]
    h = W2.shape[0]
    e_per, nnz_pad = _edge_partition(nnz)
    pad = nnz_pad - nnz

    rows = jnp.concatenate([m_indices[0], jnp.zeros((pad,), jnp.int32)])
    cols = jnp.concatenate([m_indices[1], jnp.ones((pad,), jnp.int32)])
    vals = jnp.concatenate([m_values, jnp.zeros((pad,), jnp.float32)])

    diag_all, pos_all, maxabs = _make_diag_kernel(n, e_per)(rows, cols, vals)

    y2d = _make_mlp_call(n, h, 2048)(
        diag_all, pos_all, maxabs, b.reshape(1, n), W1[1:3],
        b1.reshape(1, h), W2, b2.reshape(1, h), W3, b3.reshape(1, 1))

    p_all = _make_spmv_kernel(n, e_per)(rows, cols, vals, y2d.reshape(n))

    out2d = _make_scale_call(n)(p_all, b.reshape(1, n), y2d)
    return out2d.reshape(n)
